# baseline (speedup 1.0000x reference)
import functools

import jax
import jax.numpy as jnp
from jax import lax
from jax.experimental import pallas as pl
from jax.experimental.pallas import tpu as pltpu, tpu_sc as plsc

_NC = 2
_NS = 16
_L = 16
_NW = _NC * _NS

_N = 4096
_D = 8192
_ROWS_PER_W = _N // _NW
_BLK_ROWS = 4
_NBLK = _ROWS_PER_W // _BLK_ROWS
_SLICES = _D // _L
_UNROLL = 8
_INT_MAX = 2147483647


def _permute(x, perm):
    return lax.gather(
        x, perm.reshape(_L, 1),
        lax.GatherDimensionNumbers(
            offset_dims=(), collapsed_slice_dims=(0,), start_index_map=(0,)),
        (1,), mode=lax.GatherScatterMode.PROMISE_IN_BOUNDS)


_NCHAIN = 4


def _row_argmin(buf_ref, r):
    lane = lax.iota(jnp.int32, _L)

    def body(g, carry):
        bvs, bis, base_idx = carry
        bvs, bis = list(bvs), list(bis)
        base = g * (_UNROLL * _L)
        for u in range(_UNROLL):
            c = u % _NCHAIN
            v = buf_ref[r, pl.ds(base + u * _L, _L)]
            idx = base_idx + (u * _L)
            m = v < bvs[c]
            bvs[c] = jnp.minimum(bvs[c], v)
            bis[c] = jnp.where(m, idx, bis[c])
        return tuple(bvs), tuple(bis), base_idx + (_UNROLL * _L)

    bv0 = jnp.full((_L,), jnp.inf, jnp.float32)
    bi0 = jnp.zeros((_L,), jnp.int32)
    bvs, bis, _ = lax.fori_loop(
        0, _SLICES // _UNROLL, body,
        ((bv0,) * _NCHAIN, (bi0,) * _NCHAIN, lane))

    def merge(av, ai, cv, ci):
        pick = (cv < av) | ((cv == av) & (ci < ai))
        return jnp.where(pick, cv, av), jnp.where(pick, ci, ai)

    bv, bi = merge(*merge(bvs[0], bis[0], bvs[1], bis[1]),
                   *merge(bvs[2], bis[2], bvs[3], bis[3]))

    for s in (8, 4, 2, 1):
        perm = lane ^ s
        ov = _permute(bv, perm)
        oi = _permute(bi, perm)
        pick = (ov < bv) | ((ov == bv) & (oi < bi))
        bv = jnp.where(pick, ov, bv)
        bi = jnp.where(pick, oi, bi)
    return bi


def _wtac_body(dist_hbm, labels_hbm, out_hbm,
               buf0, buf1, amin_v, lbl_v, sem0, sem1, semg):
    wid = lax.axis_index("s") * _NC + lax.axis_index("c")
    base = wid * _ROWS_PER_W

    bufs = (buf0, buf1)
    sems = (sem0, sem1)

    def start(blk, b):
        pltpu.async_copy(
            dist_hbm.at[pl.ds(base + blk * _BLK_ROWS, _BLK_ROWS), :],
            bufs[b], sems[b])

    start(0, 0)
    start(1, 1)

    lane = lax.iota(jnp.int32, _L)
    blk_per_g = _L // _BLK_ROWS

    def super_body(g, carry):
        acc = jnp.zeros((_L,), jnp.int32)
        for b in range(blk_per_g):
            blk = blk_per_g * g + b
            buf = bufs[b % 2]
            sem = sems[b % 2]
            pltpu.make_async_copy(
                dist_hbm.at[pl.ds(0, _BLK_ROWS), :], buf, sem).wait()
            for r in range(_BLK_ROWS):
                ria = _row_argmin(buf, r)
                acc = jnp.where(lane == b * _BLK_ROWS + r, ria, acc)

            @pl.when(blk + 2 < _NBLK)
            def _start_next():
                pltpu.async_copy(
                    dist_hbm.at[pl.ds(base + (blk + 2) * _BLK_ROWS,
                                      _BLK_ROWS), :],
                    bufs[b % 2], sems[b % 2])
        amin_v[pl.ds(g * _L, _L)] = acc
        return carry

    lax.fori_loop(0, _NBLK // blk_per_g, super_body, 0)

    pltpu.async_copy(labels_hbm.at[amin_v], lbl_v, semg).wait()
    pltpu.sync_copy(lbl_v, out_hbm.at[pl.ds(base, _ROWS_PER_W)])


@jax.jit
def _wtac(distances, labels):
    mesh = plsc.VectorSubcoreMesh(core_axis_name="c", subcore_axis_name="s")
    return pl.kernel(
        _wtac_body,
        out_type=jax.ShapeDtypeStruct((_N,), jnp.int32),
        mesh=mesh,
        scratch_types=[
            pltpu.VMEM((_BLK_ROWS, _D), jnp.float32),
            pltpu.VMEM((_BLK_ROWS, _D), jnp.float32),
            pltpu.VMEM((_ROWS_PER_W,), jnp.int32),
            pltpu.VMEM((_ROWS_PER_W,), jnp.int32),
            pltpu.SemaphoreType.DMA,
            pltpu.SemaphoreType.DMA,
            pltpu.SemaphoreType.DMA,
        ],
    )(distances, labels)


def kernel(distances, labels):
    return _wtac(distances, labels.astype(jnp.int32))


# speedup vs baseline: 1.0338x; 1.0338x over previous
import functools

import jax
import jax.numpy as jnp
from jax import lax
from jax.experimental import pallas as pl
from jax.experimental.pallas import tpu as pltpu, tpu_sc as plsc

_NC = 2
_NS = 16
_L = 16
_NW = _NC * _NS

_N = 4096
_D = 8192
_ROWS_PER_W = _N // _NW
_BLK_ROWS = 4
_NBLK = _ROWS_PER_W // _BLK_ROWS
_SLICES = _D // _L
_UNROLL = 8
_INT_MAX = 2147483647


def _permute(x, perm):
    return lax.gather(
        x, perm.reshape(_L, 1),
        lax.GatherDimensionNumbers(
            offset_dims=(), collapsed_slice_dims=(0,), start_index_map=(0,)),
        (1,), mode=lax.GatherScatterMode.PROMISE_IN_BOUNDS)


def _row_argmin(buf_ref, r):
    lane = lax.iota(jnp.int32, _L)

    def body(g, carry):
        bvs, bgs, gvec = carry
        bvs, bgs = list(bvs), list(bgs)
        base = g * (_UNROLL * _L)
        for u in range(_UNROLL):
            v = buf_ref[r, pl.ds(base + u * _L, _L)]
            m = v < bvs[u]
            bvs[u] = jnp.minimum(bvs[u], v)
            bgs[u] = jnp.where(m, gvec, bgs[u])
        return tuple(bvs), tuple(bgs), gvec + 1

    bv0 = jnp.full((_L,), jnp.inf, jnp.float32)
    bg0 = jnp.zeros((_L,), jnp.int32)
    gv0 = jnp.zeros((_L,), jnp.int32)
    bvs, bgs, _ = lax.fori_loop(
        0, _SLICES // _UNROLL, body,
        ((bv0,) * _UNROLL, (bg0,) * _UNROLL, gv0))

    def merge(av, ai, cv, ci):
        pick = (cv < av) | ((cv == av) & (ci < ai))
        return jnp.where(pick, cv, av), jnp.where(pick, ci, ai)

    bv = bi = None
    for c in range(_UNROLL):
        idx_c = bgs[c] * (_UNROLL * _L) + (lane + c * _L)
        if bv is None:
            bv, bi = bvs[c], idx_c
        else:
            bv, bi = merge(bv, bi, bvs[c], idx_c)

    for s in (8, 4, 2, 1):
        perm = lane ^ s
        ov = _permute(bv, perm)
        oi = _permute(bi, perm)
        pick = (ov < bv) | ((ov == bv) & (oi < bi))
        bv = jnp.where(pick, ov, bv)
        bi = jnp.where(pick, oi, bi)
    return bi


def _wtac_body(dist_hbm, labels_hbm, out_hbm,
               buf0, buf1, amin_v, lbl_v, sem0, sem1, semg):
    wid = lax.axis_index("s") * _NC + lax.axis_index("c")
    base = wid * _ROWS_PER_W

    bufs = (buf0, buf1)
    sems = (sem0, sem1)

    def start(blk, b):
        pltpu.async_copy(
            dist_hbm.at[pl.ds(base + blk * _BLK_ROWS, _BLK_ROWS), :],
            bufs[b], sems[b])

    start(0, 0)
    start(1, 1)

    lane = lax.iota(jnp.int32, _L)
    blk_per_g = _L // _BLK_ROWS

    def super_body(g, carry):
        acc = jnp.zeros((_L,), jnp.int32)
        for b in range(blk_per_g):
            blk = blk_per_g * g + b
            buf = bufs[b % 2]
            sem = sems[b % 2]
            pltpu.make_async_copy(
                dist_hbm.at[pl.ds(0, _BLK_ROWS), :], buf, sem).wait()
            for r in range(_BLK_ROWS):
                ria = _row_argmin(buf, r)
                acc = jnp.where(lane == b * _BLK_ROWS + r, ria, acc)

            @pl.when(blk + 2 < _NBLK)
            def _start_next():
                pltpu.async_copy(
                    dist_hbm.at[pl.ds(base + (blk + 2) * _BLK_ROWS,
                                      _BLK_ROWS), :],
                    bufs[b % 2], sems[b % 2])
        amin_v[pl.ds(g * _L, _L)] = acc
        return carry

    lax.fori_loop(0, _NBLK // blk_per_g, super_body, 0)

    pltpu.async_copy(labels_hbm.at[amin_v], lbl_v, semg).wait()
    pltpu.sync_copy(lbl_v, out_hbm.at[pl.ds(base, _ROWS_PER_W)])


@jax.jit
def _wtac(distances, labels):
    mesh = plsc.VectorSubcoreMesh(core_axis_name="c", subcore_axis_name="s")
    return pl.kernel(
        _wtac_body,
        out_type=jax.ShapeDtypeStruct((_N,), jnp.int32),
        mesh=mesh,
        scratch_types=[
            pltpu.VMEM((_BLK_ROWS, _D), jnp.float32),
            pltpu.VMEM((_BLK_ROWS, _D), jnp.float32),
            pltpu.VMEM((_ROWS_PER_W,), jnp.int32),
            pltpu.VMEM((_ROWS_PER_W,), jnp.int32),
            pltpu.SemaphoreType.DMA,
            pltpu.SemaphoreType.DMA,
            pltpu.SemaphoreType.DMA,
        ],
    )(distances, labels)


def kernel(distances, labels):
    return _wtac(distances, labels.astype(jnp.int32))


# speedup vs baseline: 1.2848x; 1.2428x over previous
import functools

import jax
import jax.numpy as jnp
from jax import lax
from jax.experimental import pallas as pl
from jax.experimental.pallas import tpu as pltpu, tpu_sc as plsc

_NC = 2
_NS = 16
_L = 16
_NW = _NC * _NS

_N = 4096
_D = 8192

_N_SC = 2048
_N_TC = _N - _N_SC

_ROWS_PER_W = _N_SC // _NW
_BLK_ROWS = 4
_NBLK = _ROWS_PER_W // _BLK_ROWS
_SLICES = _D // _L
_UNROLL = 8
_INT_MAX = 2147483647

_TC_BLK = 256
_LBL_BITS = 13


def _permute(x, perm):
    return lax.gather(
        x, perm.reshape(_L, 1),
        lax.GatherDimensionNumbers(
            offset_dims=(), collapsed_slice_dims=(0,), start_index_map=(0,)),
        (1,), mode=lax.GatherScatterMode.PROMISE_IN_BOUNDS)


def _row_argmin(buf_ref, r):
    lane = lax.iota(jnp.int32, _L)

    def body(g, carry):
        bvs, bgs, gvec = carry
        bvs, bgs = list(bvs), list(bgs)
        base = g * (_UNROLL * _L)
        for u in range(_UNROLL):
            v = buf_ref[r, pl.ds(base + u * _L, _L)]
            m = v < bvs[u]
            bvs[u] = jnp.minimum(bvs[u], v)
            bgs[u] = jnp.where(m, gvec, bgs[u])
        return tuple(bvs), tuple(bgs), gvec + 1

    bv0 = jnp.full((_L,), jnp.inf, jnp.float32)
    bg0 = jnp.zeros((_L,), jnp.int32)
    gv0 = jnp.zeros((_L,), jnp.int32)
    bvs, bgs, _ = lax.fori_loop(
        0, _SLICES // _UNROLL, body,
        ((bv0,) * _UNROLL, (bg0,) * _UNROLL, gv0))

    def merge(av, ai, cv, ci):
        pick = (cv < av) | ((cv == av) & (ci < ai))
        return jnp.where(pick, cv, av), jnp.where(pick, ci, ai)

    bv = bi = None
    for c in range(_UNROLL):
        idx_c = bgs[c] * (_UNROLL * _L) + (lane + c * _L)
        if bv is None:
            bv, bi = bvs[c], idx_c
        else:
            bv, bi = merge(bv, bi, bvs[c], idx_c)

    for s in (8, 4, 2, 1):
        perm = lane ^ s
        ov = _permute(bv, perm)
        oi = _permute(bi, perm)
        pick = (ov < bv) | ((ov == bv) & (oi < bi))
        bv = jnp.where(pick, ov, bv)
        bi = jnp.where(pick, oi, bi)
    return bi


def _wtac_sc_body(dist_hbm, labels_hbm, out_hbm,
                  buf0, buf1, amin_v, lbl_v, sem0, sem1, semg):
    wid = lax.axis_index("s") * _NC + lax.axis_index("c")
    base = wid * _ROWS_PER_W

    bufs = (buf0, buf1)
    sems = (sem0, sem1)

    def start(blk, b):
        pltpu.async_copy(
            dist_hbm.at[pl.ds(base + blk * _BLK_ROWS, _BLK_ROWS), :],
            bufs[b], sems[b])

    start(0, 0)
    start(1, 1)

    lane = lax.iota(jnp.int32, _L)
    blk_per_g = _L // _BLK_ROWS

    def super_body(g, carry):
        acc = jnp.zeros((_L,), jnp.int32)
        for b in range(blk_per_g):
            blk = blk_per_g * g + b
            buf = bufs[b % 2]
            sem = sems[b % 2]
            pltpu.make_async_copy(
                dist_hbm.at[pl.ds(0, _BLK_ROWS), :], buf, sem).wait()
            for r in range(_BLK_ROWS):
                ria = _row_argmin(buf, r)
                acc = jnp.where(lane == b * _BLK_ROWS + r, ria, acc)

            @pl.when(blk + 2 < _NBLK)
            def _start_next():
                pltpu.async_copy(
                    dist_hbm.at[pl.ds(base + (blk + 2) * _BLK_ROWS,
                                      _BLK_ROWS), :],
                    bufs[b % 2], sems[b % 2])
        amin_v[pl.ds(g * _L, _L)] = acc
        return carry

    lax.fori_loop(0, _NBLK // blk_per_g, super_body, 0)

    pltpu.async_copy(labels_hbm.at[amin_v], lbl_v, semg).wait()
    pltpu.sync_copy(lbl_v, out_hbm.at[pl.ds(base, _ROWS_PER_W)])


def _wtac_tc_body(labels_ref, dist_ref, out_ref):
    v = dist_ref[...]
    lbl = labels_ref[...]
    rowmin = jnp.min(v, axis=1, keepdims=True)
    col = lax.broadcasted_iota(jnp.int32, (_TC_BLK, _D), 1)
    packed = (col << _LBL_BITS) | lbl[None, :]
    win = jnp.min(jnp.where(v == rowmin, packed, _INT_MAX), axis=1)
    out_ref[...] = win & ((1 << _LBL_BITS) - 1)


@jax.jit
def _wtac(distances, labels):
    mesh = plsc.VectorSubcoreMesh(core_axis_name="c", subcore_axis_name="s")
    sc_out = pl.kernel(
        _wtac_sc_body,
        out_type=jax.ShapeDtypeStruct((_N_SC,), jnp.int32),
        mesh=mesh,
        scratch_types=[
            pltpu.VMEM((_BLK_ROWS, _D), jnp.float32),
            pltpu.VMEM((_BLK_ROWS, _D), jnp.float32),
            pltpu.VMEM((_ROWS_PER_W,), jnp.int32),
            pltpu.VMEM((_ROWS_PER_W,), jnp.int32),
            pltpu.SemaphoreType.DMA,
            pltpu.SemaphoreType.DMA,
            pltpu.SemaphoreType.DMA,
        ],
    )(distances, labels)

    tc_out = pl.pallas_call(
        _wtac_tc_body,
        grid=(_N_TC // _TC_BLK,),
        in_specs=[
            pl.BlockSpec((_D,), lambda g: (0,)),
            pl.BlockSpec((_TC_BLK, _D), lambda g: (g + _N_SC // _TC_BLK, 0)),
        ],
        out_specs=pl.BlockSpec((_TC_BLK,), lambda g: (g,)),
        out_shape=jax.ShapeDtypeStruct((_N_TC,), jnp.int32),
    )(labels, distances)

    return jnp.concatenate([sc_out, tc_out])


def kernel(distances, labels):
    return _wtac(distances, labels.astype(jnp.int32))
